# baseline (speedup 1.0000x reference)
import functools

import jax
import jax.numpy as jnp
from jax import lax
from jax.experimental import pallas as pl
from jax.experimental.pallas import tpu as pltpu
from jax.experimental.pallas import tpu_sc as plsc

T = 12
N = 50000
E = 800000
NP = 50176
RT = NP // 16
EP = 802816
EC = EP // 128
CAT_C = 4 * NP // 128
BN = 256
NB = NP // BN

_mesh = plsc.VectorSubcoreMesh(core_axis_name="c", subcore_axis_name="s")
_sc_params = pltpu.CompilerParams(use_tc_tiling_on_sc=False)


def _sc_embed_deg(cat_b, emb_all, dst_b, ones_deg, zdeg, th):
  @functools.partial(
      pl.kernel,
      out_type=[
          jax.ShapeDtypeStruct((NP * 4 * th, 16), jnp.float32),
          jax.ShapeDtypeStruct((th, 2, NP, 1), jnp.float32),
      ],
      mesh=_mesh,
      scratch_types=[
          pltpu.VMEM((128,), jnp.int32),
          pltpu.VMEM((128, 16), jnp.float32),
          pltpu.VMEM((1, 128), jnp.int32),
          pltpu.VMEM((128, 1), jnp.float32),
          pltpu.VMEM_SHARED((NP, 1), jnp.float32),
      ],
      compiler_params=_sc_params,
  )
  def k(cat_hbm, emb_hbm, dst_hbm, ones_hbm, zdeg_hbm, x4_out, deg_out,
        eidx_v, erows_v, didx_v, ones_v, dacc_sh):
    c = lax.axis_index("c")
    s = lax.axis_index("s")
    w = c * 16 + s
    pltpu.sync_copy(ones_hbm, ones_v)
    emb_chunks = CAT_C // 32
    deg_chunks = EC // 32
    for t in range(th):
      @pl.loop(0, emb_chunks)
      def _(j):
        ch = w * emb_chunks + j
        pltpu.sync_copy(cat_hbm.at[t, ch], eidx_v)
        pltpu.sync_copy(emb_hbm.at[eidx_v], erows_v)
        pltpu.sync_copy(erows_v,
                        x4_out.at[pl.ds((t * CAT_C + ch) * 128, 128)])

      pltpu.sync_copy(zdeg_hbm, dacc_sh.at[pl.ds(s * RT, RT)])
      plsc.subcore_barrier()

      @pl.loop(0, deg_chunks)
      def _(j):
        ch = (c * 16 + s) * deg_chunks + j
        pltpu.sync_copy(dst_hbm.at[t, ch], didx_v.at[0])
        pltpu.sync_copy(ones_v, dacc_sh.at[didx_v.at[0]], add=True)

      plsc.subcore_barrier()
      pltpu.sync_copy(dacc_sh.at[pl.ds(s * RT, RT)],
                      deg_out.at[t, c, pl.ds(s * RT, RT)])

  return k(cat_b, emb_all, dst_b, ones_deg, zdeg)


def _sc_scatter(y_b, src_b, dst_b, zconv, cb_total, th):
  passes = cb_total // 2
  conv_chunks = EC // 16
  G = 2
  NBUF = 3
  GROUPS = conv_chunks // G

  @functools.partial(
      pl.kernel,
      out_type=jax.ShapeDtypeStruct((th, cb_total, NP, 32), jnp.float32),
      mesh=_mesh,
      scratch_types=[
          pltpu.VMEM((NBUF, G, 128), jnp.int32),
          pltpu.VMEM((NBUF, G, 128), jnp.int32),
          pltpu.VMEM((NBUF, G, 128, 32), jnp.float32),
          pltpu.VMEM_SHARED((NP, 32), jnp.float32),
          [pltpu.SemaphoreType.DMA] * NBUF,
          [pltpu.SemaphoreType.DMA] * NBUF,
      ],
      compiler_params=_sc_params,
  )
  def k(y_hbm, src_hbm, dst_hbm, zc_hbm, scat_out, sidx_v, didx_v, rows_v,
        acc_sh, semg, sems):
    c = lax.axis_index("c")
    s = lax.axis_index("s")

    def load_and_fire(t, cb, g, b):
      base = s * conv_chunks + g * G
      pltpu.sync_copy(src_hbm.at[t, pl.ds(base, G)], sidx_v.at[b])
      pltpu.sync_copy(dst_hbm.at[t, pl.ds(base, G)], didx_v.at[b])
      for j in range(G):
        pltpu.async_copy(y_hbm.at[t, cb].at[sidx_v.at[b, j]],
                         rows_v.at[b, j], semg[b])

    def gather_to_scatter(t, cb, b):
      for j in range(G):
        pltpu.make_async_copy(y_hbm.at[t, cb].at[sidx_v.at[b, j]],
                              rows_v.at[b, j], semg[b]).wait()
      for j in range(G):
        pltpu.async_copy(rows_v.at[b, j], acc_sh.at[didx_v.at[b, j]],
                         sems[b], add=True)

    def drain_scatter(b):
      for j in range(G):
        pltpu.make_async_copy(rows_v.at[b, j], acc_sh.at[didx_v.at[b, j]],
                              sems[b]).wait()

    for t in range(th):
      for p in range(passes):
        cb = c * passes + p
        pltpu.sync_copy(zc_hbm, acc_sh.at[pl.ds(s * RT, RT)])
        plsc.subcore_barrier()

        for b in range(NBUF):
          load_and_fire(t, cb, b, b)

        @pl.loop(0, GROUPS, step=NBUF)
        def _(i):
          for b in range(NBUF):
            @pl.when(i + b < GROUPS)
            def _():
              gather_to_scatter(t, cb, b)

          for b in range(NBUF):
            @pl.when(i + b + NBUF < GROUPS)
            def _():
              drain_scatter(b)
              load_and_fire(t, cb, i + b + NBUF, b)

        for b in range(NBUF):
          drain_scatter(b)
        plsc.subcore_barrier()
        pltpu.sync_copy(acc_sh.at[pl.ds(s * RT, RT)],
                        scat_out.at[t, cb, pl.ds(s * RT, RT)])

  return k(y_b, src_b, dst_b, zconv)


def _mm1(xcat, num_p, deg2, W1, th):
  def body(xcat_ref, num_ref, deg_ref, w1_ref, y1_ref):
    deg = deg_ref[0, 0, :, 0] + deg_ref[0, 1, :, 0] + 1.0
    dis = lax.rsqrt(deg)
    xw = jnp.dot(xcat_ref[0], w1_ref[0:64],
                 preferred_element_type=jnp.float32)
    xw += jnp.dot(num_ref[0], w1_ref[64:96],
                  preferred_element_type=jnp.float32)
    y = xw * dis[:, None]
    for cb in range(4):
      y1_ref[0, cb] = y[:, 32 * cb:32 * (cb + 1)]

  return pl.pallas_call(
      body,
      grid=(th, NB),
      in_specs=[
          pl.BlockSpec((1, BN, 64), lambda t, n: (t, n, 0)),
          pl.BlockSpec((1, BN, 32), lambda t, n: (t, n, 0)),
          pl.BlockSpec((1, 2, BN, 1), lambda t, n: (t, 0, n, 0)),
          pl.BlockSpec((96, 128), lambda t, n: (0, 0)),
      ],
      out_specs=pl.BlockSpec((1, 4, BN, 32), lambda t, n: (t, 0, n, 0)),
      out_shape=jax.ShapeDtypeStruct((th, 4, NP, 32), jnp.float32),
      compiler_params=pltpu.CompilerParams(
          dimension_semantics=("parallel", "parallel")),
  )(xcat, num_p, deg2, W1)


def _post1_mm2(scat1, y1, deg2, W2, b1, th):
  def body(scat_ref, y1_ref, deg_ref, w2_ref, b1_ref, y2_ref):
    deg = deg_ref[0, 0, :, 0] + deg_ref[0, 1, :, 0] + 1.0
    dis = lax.rsqrt(deg)[:, None]
    h = jnp.concatenate(
        [scat_ref[0, i] + y1_ref[0, i] for i in range(4)], axis=1)
    h1 = jnp.maximum(h * dis + b1_ref[0], 0.0)
    y2 = jnp.dot(h1, w2_ref[...], preferred_element_type=jnp.float32) * dis
    for i in range(2):
      y2_ref[0, i] = y2[:, 32 * i:32 * (i + 1)]

  return pl.pallas_call(
      body,
      grid=(th, NB),
      in_specs=[
          pl.BlockSpec((1, 4, BN, 32), lambda t, n: (t, 0, n, 0)),
          pl.BlockSpec((1, 4, BN, 32), lambda t, n: (t, 0, n, 0)),
          pl.BlockSpec((1, 2, BN, 1), lambda t, n: (t, 0, n, 0)),
          pl.BlockSpec((128, 64), lambda t, n: (0, 0)),
          pl.BlockSpec((1, 128), lambda t, n: (0, 0)),
      ],
      out_specs=pl.BlockSpec((1, 2, BN, 32), lambda t, n: (t, 0, n, 0)),
      out_shape=jax.ShapeDtypeStruct((th, 2, NP, 32), jnp.float32),
      compiler_params=pltpu.CompilerParams(
          dimension_semantics=("parallel", "parallel")),
  )(scat1, y1, deg2, W2, b1)


def _post2_gru(scat2a, scat2b, y2a, y2b, deg2a, deg2b, b2, W_ihT, W_hhT,
               b_ih, b_hh, th):
  def body(scat_a, scat_b, y2_a, y2_b, deg_a, deg_b, b2_ref, wih_ref,
           whh_ref, bih_ref, bhh_ref, h_ref):
    h = jnp.zeros((BN, 64), jnp.float32)
    for t in range(2 * th):
      scat_ref, y2_ref, deg_ref = (
          (scat_a, y2_a, deg_a) if t < th else (scat_b, y2_b, deg_b))
      tt = t if t < th else t - th
      deg = deg_ref[tt, 0, :, 0] + deg_ref[tt, 1, :, 0] + 1.0
      dis = lax.rsqrt(deg)[:, None]
      x = jnp.concatenate(
          [scat_ref[tt, i] + y2_ref[tt, i] for i in range(2)], axis=1)
      x = x * dis + b2_ref[0]
      gi = jnp.dot(x, wih_ref[...],
                   preferred_element_type=jnp.float32) + bih_ref[0]
      gh = jnp.dot(h, whh_ref[...],
                   preferred_element_type=jnp.float32) + bhh_ref[0]
      r = jax.nn.sigmoid(gi[:, 0:64] + gh[:, 0:64])
      z = jax.nn.sigmoid(gi[:, 64:128] + gh[:, 64:128])
      n_ = jnp.tanh(gi[:, 128:192] + r * gh[:, 128:192])
      h = (1.0 - z) * n_ + z * h
    h_ref[...] = h

  half = [
      pl.BlockSpec((th, 2, BN, 32), lambda n: (0, 0, n, 0)),
      pl.BlockSpec((th, 2, BN, 32), lambda n: (0, 0, n, 0)),
  ]
  return pl.pallas_call(
      body,
      grid=(NB,),
      in_specs=half + half[:1] * 0 + [
          pl.BlockSpec((th, 2, BN, 32), lambda n: (0, 0, n, 0)),
          pl.BlockSpec((th, 2, BN, 32), lambda n: (0, 0, n, 0)),
          pl.BlockSpec((th, 2, BN, 1), lambda n: (0, 0, n, 0)),
          pl.BlockSpec((th, 2, BN, 1), lambda n: (0, 0, n, 0)),
          pl.BlockSpec((1, 64), lambda n: (0, 0)),
          pl.BlockSpec((64, 192), lambda n: (0, 0)),
          pl.BlockSpec((64, 192), lambda n: (0, 0)),
          pl.BlockSpec((1, 192), lambda n: (0, 0)),
          pl.BlockSpec((1, 192), lambda n: (0, 0)),
      ],
      out_specs=pl.BlockSpec((BN, 64), lambda n: (n, 0)),
      out_shape=jax.ShapeDtypeStruct((NP, 64), jnp.float32),
      compiler_params=pltpu.CompilerParams(
          dimension_semantics=("parallel",)),
  )(scat2a, scat2b, y2a, y2b, deg2a, deg2b, b2, W_ihT, W_hhT, b_ih, b_hh)


def kernel(cat_x, num_x, edges, emb0, emb1, emb2, emb3, W1, b1, W2, b2,
           W_ih, W_hh, b_ih, b_hh):
  f32 = jnp.float32
  i32 = jnp.int32

  emb_all = jnp.concatenate([emb0, emb1, emb2, emb3], axis=0)
  offs = jnp.array([0, 1000, 2000, 3000], i32)
  cat_o = cat_x + offs[None, None, :]
  cat_o = jnp.pad(cat_o, ((0, 0), (0, NP - N), (0, 0)))
  cat_b = cat_o.reshape(T, CAT_C, 128)

  pad_src = jnp.broadcast_to(
      (jnp.arange(EP - E) % 128).astype(i32), (T, EP - E))
  pad_dst = jnp.broadcast_to(
      (N + jnp.arange(EP - E) % (NP - N)).astype(i32), (T, EP - E))
  src_b = jnp.concatenate([edges[:, 0], pad_src], axis=1).reshape(T, EC, 128)
  dst_b = jnp.concatenate([edges[:, 1], pad_dst], axis=1).reshape(T, EC, 128)

  num_p = jnp.pad(num_x, ((0, 0), (0, NP - N), (0, 0)))
  ones_deg = jnp.ones((128, 1), f32)
  zdeg = jnp.zeros((RT, 1), f32)
  zconv = jnp.zeros((RT, 32), f32)

  th = T // 2
  xcat, y1, scat1, y2, scat2, deg = {}, {}, {}, {}, {}, {}
  sl = {0: slice(0, th), 1: slice(th, T)}
  for i in (0, 1):
    x4_flat, deg[i] = _sc_embed_deg(cat_b[sl[i]], emb_all, dst_b[sl[i]],
                                    ones_deg, zdeg, th)
    xcat[i] = x4_flat.reshape(th, NP, 64)
  for i in (0, 1):
    y1[i] = _mm1(xcat[i], num_p[sl[i]], deg[i], W1, th)
  for i in (0, 1):
    scat1[i] = _sc_scatter(y1[i], src_b[sl[i]], dst_b[sl[i]], zconv, 4, th)
    y2[i] = _post1_mm2(scat1[i], y1[i], deg[i], W2, b1.reshape(1, 128), th)
  for i in (0, 1):
    scat2[i] = _sc_scatter(y2[i], src_b[sl[i]], dst_b[sl[i]], zconv, 2, th)
  h = _post2_gru(scat2[0], scat2[1], y2[0], y2[1], deg[0], deg[1],
                 b2.reshape(1, 64), W_ih.T, W_hh.T,
                 b_ih.reshape(1, 192), b_hh.reshape(1, 192), th)
  return h[:N]


# speedup vs baseline: 1.0530x; 1.0530x over previous
import functools

import jax
import jax.numpy as jnp
from jax import lax
from jax.experimental import pallas as pl
from jax.experimental.pallas import tpu as pltpu
from jax.experimental.pallas import tpu_sc as plsc

T = 12
N = 50000
E = 800000
NP = 50176
RT = NP // 16
EP = 802816
EC = EP // 128
CAT_C = 4 * NP // 128
BN = 256
NB = NP // BN

_mesh = plsc.VectorSubcoreMesh(core_axis_name="c", subcore_axis_name="s")
_sc_params = pltpu.CompilerParams(use_tc_tiling_on_sc=False,
                                  needs_layout_passes=False)


def _sc_embed_deg(cat_b, emb_all, dst_b, th):
  DG = 7

  @functools.partial(
      pl.kernel,
      out_type=[
          jax.ShapeDtypeStruct((NP * 4 * th, 16), jnp.float32),
          jax.ShapeDtypeStruct((th, 32, NP), jnp.float32),
      ],
      mesh=_mesh,
      scratch_types=[
          pltpu.VMEM((128,), jnp.int32),
          pltpu.VMEM((128, 16), jnp.float32),
          pltpu.VMEM((DG, 128), jnp.int32),
          pltpu.VMEM((NP,), jnp.float32),
      ],
      compiler_params=_sc_params,
  )
  def k(cat_hbm, emb_hbm, dst_hbm, x4_out, deg_out,
        eidx_v, erows_v, didx_v, dloc_v):
    c = lax.axis_index("c")
    s = lax.axis_index("s")
    w = c * 16 + s
    ones16 = jnp.full((16,), 1.0, jnp.float32)
    zeros16 = jnp.zeros((16,), jnp.float32)
    emb_chunks = CAT_C // 32
    deg_chunks = EC // 32
    for t in range(th):
      @pl.loop(0, emb_chunks)
      def _(j):
        ch = w * emb_chunks + j
        pltpu.sync_copy(cat_hbm.at[t, ch], eidx_v)
        pltpu.sync_copy(emb_hbm.at[eidx_v], erows_v)
        pltpu.sync_copy(erows_v,
                        x4_out.at[pl.ds((t * CAT_C + ch) * 128, 128)])

      @pl.loop(0, NP // 16)
      def _(r):
        dloc_v[pl.ds(r * 16, 16)] = zeros16

      @pl.loop(0, deg_chunks // DG)
      def _(g):
        ch = w * deg_chunks + g * DG
        pltpu.sync_copy(dst_hbm.at[t, pl.ds(ch, DG)], didx_v)
        for jj in range(DG):
          for kk in range(8):
            idx = didx_v[jj, pl.ds(kk * 16, 16)]
            plsc.addupdate_scatter(dloc_v, [idx], ones16)

      pltpu.sync_copy(dloc_v, deg_out.at[t, w])

  return k(cat_b, emb_all, dst_b)


def _sc_scatter(y_b, src_b, dst_b, zconv, cb_total, th):
  passes = cb_total // 2
  conv_chunks = EC // 16
  G = 2
  NBUF = 3
  GROUPS = conv_chunks // G

  @functools.partial(
      pl.kernel,
      out_type=jax.ShapeDtypeStruct((th, cb_total, NP, 32), jnp.float32),
      mesh=_mesh,
      scratch_types=[
          pltpu.VMEM((NBUF, G, 128), jnp.int32),
          pltpu.VMEM((NBUF, G, 128), jnp.int32),
          pltpu.VMEM((NBUF, G, 128, 32), jnp.float32),
          pltpu.VMEM_SHARED((NP, 32), jnp.float32),
          [pltpu.SemaphoreType.DMA] * NBUF,
          [pltpu.SemaphoreType.DMA] * NBUF,
      ],
      compiler_params=_sc_params,
  )
  def k(y_hbm, src_hbm, dst_hbm, zc_hbm, scat_out, sidx_v, didx_v, rows_v,
        acc_sh, semg, sems):
    c = lax.axis_index("c")
    s = lax.axis_index("s")

    def load_and_fire(t, cb, g, b):
      base = s * conv_chunks + g * G
      pltpu.sync_copy(src_hbm.at[t, pl.ds(base, G)], sidx_v.at[b])
      pltpu.sync_copy(dst_hbm.at[t, pl.ds(base, G)], didx_v.at[b])
      for j in range(G):
        pltpu.async_copy(y_hbm.at[t, cb].at[sidx_v.at[b, j]],
                         rows_v.at[b, j], semg[b])

    def gather_to_scatter(t, cb, b):
      for j in range(G):
        pltpu.make_async_copy(y_hbm.at[t, cb].at[sidx_v.at[b, j]],
                              rows_v.at[b, j], semg[b]).wait()
      for j in range(G):
        pltpu.async_copy(rows_v.at[b, j], acc_sh.at[didx_v.at[b, j]],
                         sems[b], add=True)

    def drain_scatter(b):
      for j in range(G):
        pltpu.make_async_copy(rows_v.at[b, j], acc_sh.at[didx_v.at[b, j]],
                              sems[b]).wait()

    for t in range(th):
      for p in range(passes):
        cb = c * passes + p
        pltpu.sync_copy(zc_hbm, acc_sh.at[pl.ds(s * RT, RT)])
        plsc.subcore_barrier()

        for b in range(NBUF):
          load_and_fire(t, cb, b, b)

        @pl.loop(0, GROUPS, step=NBUF)
        def _(i):
          for b in range(NBUF):
            @pl.when(i + b < GROUPS)
            def _():
              gather_to_scatter(t, cb, b)

          for b in range(NBUF):
            @pl.when(i + b + NBUF < GROUPS)
            def _():
              drain_scatter(b)
              load_and_fire(t, cb, i + b + NBUF, b)

        for b in range(NBUF):
          drain_scatter(b)
        plsc.subcore_barrier()
        pltpu.sync_copy(acc_sh.at[pl.ds(s * RT, RT)],
                        scat_out.at[t, cb, pl.ds(s * RT, RT)])

  return k(y_b, src_b, dst_b, zconv)


def _mm1(xcat, num_p, deg2, W1, th):
  def body(xcat_ref, num_ref, deg_ref, w1_ref, y1_ref):
    deg = jnp.sum(deg_ref[0], axis=0) + 1.0
    dis = lax.rsqrt(deg)
    xw = jnp.dot(xcat_ref[0], w1_ref[0:64],
                 preferred_element_type=jnp.float32)
    xw += jnp.dot(num_ref[0], w1_ref[64:96],
                  preferred_element_type=jnp.float32)
    y = xw * dis[:, None]
    for cb in range(4):
      y1_ref[0, cb] = y[:, 32 * cb:32 * (cb + 1)]

  return pl.pallas_call(
      body,
      grid=(th, NB),
      in_specs=[
          pl.BlockSpec((1, BN, 64), lambda t, n: (t, n, 0)),
          pl.BlockSpec((1, BN, 32), lambda t, n: (t, n, 0)),
          pl.BlockSpec((1, 32, BN), lambda t, n: (t, 0, n)),
          pl.BlockSpec((96, 128), lambda t, n: (0, 0)),
      ],
      out_specs=pl.BlockSpec((1, 4, BN, 32), lambda t, n: (t, 0, n, 0)),
      out_shape=jax.ShapeDtypeStruct((th, 4, NP, 32), jnp.float32),
      compiler_params=pltpu.CompilerParams(
          dimension_semantics=("parallel", "parallel")),
  )(xcat, num_p, deg2, W1)


def _post1_mm2(scat1, y1, deg2, W2, b1, th):
  def body(scat_ref, y1_ref, deg_ref, w2_ref, b1_ref, y2_ref):
    deg = jnp.sum(deg_ref[0], axis=0) + 1.0
    dis = lax.rsqrt(deg)[:, None]
    h = jnp.concatenate(
        [scat_ref[0, i] + y1_ref[0, i] for i in range(4)], axis=1)
    h1 = jnp.maximum(h * dis + b1_ref[0], 0.0)
    y2 = jnp.dot(h1, w2_ref[...], preferred_element_type=jnp.float32) * dis
    for i in range(2):
      y2_ref[0, i] = y2[:, 32 * i:32 * (i + 1)]

  return pl.pallas_call(
      body,
      grid=(th, NB),
      in_specs=[
          pl.BlockSpec((1, 4, BN, 32), lambda t, n: (t, 0, n, 0)),
          pl.BlockSpec((1, 4, BN, 32), lambda t, n: (t, 0, n, 0)),
          pl.BlockSpec((1, 32, BN), lambda t, n: (t, 0, n)),
          pl.BlockSpec((128, 64), lambda t, n: (0, 0)),
          pl.BlockSpec((1, 128), lambda t, n: (0, 0)),
      ],
      out_specs=pl.BlockSpec((1, 2, BN, 32), lambda t, n: (t, 0, n, 0)),
      out_shape=jax.ShapeDtypeStruct((th, 2, NP, 32), jnp.float32),
      compiler_params=pltpu.CompilerParams(
          dimension_semantics=("parallel", "parallel")),
  )(scat1, y1, deg2, W2, b1)


def _post2_gru(scat2a, scat2b, y2a, y2b, deg2a, deg2b, b2, W_ihT, W_hhT,
               b_ih, b_hh, th):
  def body(scat_a, scat_b, y2_a, y2_b, deg_a, deg_b, b2_ref, wih_ref,
           whh_ref, bih_ref, bhh_ref, h_ref):
    h = jnp.zeros((BN, 64), jnp.float32)
    for t in range(2 * th):
      scat_ref, y2_ref, deg_ref = (
          (scat_a, y2_a, deg_a) if t < th else (scat_b, y2_b, deg_b))
      tt = t if t < th else t - th
      deg = jnp.sum(deg_ref[tt], axis=0) + 1.0
      dis = lax.rsqrt(deg)[:, None]
      x = jnp.concatenate(
          [scat_ref[tt, i] + y2_ref[tt, i] for i in range(2)], axis=1)
      x = x * dis + b2_ref[0]
      gi = jnp.dot(x, wih_ref[...],
                   preferred_element_type=jnp.float32) + bih_ref[0]
      gh = jnp.dot(h, whh_ref[...],
                   preferred_element_type=jnp.float32) + bhh_ref[0]
      r = jax.nn.sigmoid(gi[:, 0:64] + gh[:, 0:64])
      z = jax.nn.sigmoid(gi[:, 64:128] + gh[:, 64:128])
      n_ = jnp.tanh(gi[:, 128:192] + r * gh[:, 128:192])
      h = (1.0 - z) * n_ + z * h
    h_ref[...] = h

  half = [
      pl.BlockSpec((th, 2, BN, 32), lambda n: (0, 0, n, 0)),
      pl.BlockSpec((th, 2, BN, 32), lambda n: (0, 0, n, 0)),
  ]
  return pl.pallas_call(
      body,
      grid=(NB,),
      in_specs=half + half[:1] * 0 + [
          pl.BlockSpec((th, 2, BN, 32), lambda n: (0, 0, n, 0)),
          pl.BlockSpec((th, 2, BN, 32), lambda n: (0, 0, n, 0)),
          pl.BlockSpec((th, 32, BN), lambda n: (0, 0, n)),
          pl.BlockSpec((th, 32, BN), lambda n: (0, 0, n)),
          pl.BlockSpec((1, 64), lambda n: (0, 0)),
          pl.BlockSpec((64, 192), lambda n: (0, 0)),
          pl.BlockSpec((64, 192), lambda n: (0, 0)),
          pl.BlockSpec((1, 192), lambda n: (0, 0)),
          pl.BlockSpec((1, 192), lambda n: (0, 0)),
      ],
      out_specs=pl.BlockSpec((BN, 64), lambda n: (n, 0)),
      out_shape=jax.ShapeDtypeStruct((NP, 64), jnp.float32),
      compiler_params=pltpu.CompilerParams(
          dimension_semantics=("parallel",)),
  )(scat2a, scat2b, y2a, y2b, deg2a, deg2b, b2, W_ihT, W_hhT, b_ih, b_hh)


def kernel(cat_x, num_x, edges, emb0, emb1, emb2, emb3, W1, b1, W2, b2,
           W_ih, W_hh, b_ih, b_hh):
  f32 = jnp.float32
  i32 = jnp.int32

  emb_all = jnp.concatenate([emb0, emb1, emb2, emb3], axis=0)
  offs = jnp.array([0, 1000, 2000, 3000], i32)
  cat_o = cat_x + offs[None, None, :]
  cat_o = jnp.pad(cat_o, ((0, 0), (0, NP - N), (0, 0)))
  cat_b = cat_o.reshape(T, CAT_C, 128)

  pad_src = jnp.broadcast_to(
      (jnp.arange(EP - E) % 128).astype(i32), (T, EP - E))
  pad_dst = jnp.broadcast_to(
      (N + jnp.arange(EP - E) % (NP - N)).astype(i32), (T, EP - E))
  src_b = jnp.concatenate([edges[:, 0], pad_src], axis=1).reshape(T, EC, 128)
  dst_b = jnp.concatenate([edges[:, 1], pad_dst], axis=1).reshape(T, EC, 128)

  num_p = jnp.pad(num_x, ((0, 0), (0, NP - N), (0, 0)))
  zconv = jnp.zeros((RT, 32), f32)

  th = T // 2
  xcat, y1, scat1, y2, scat2, deg = {}, {}, {}, {}, {}, {}
  sl = {0: slice(0, th), 1: slice(th, T)}
  for i in (0, 1):
    x4_flat, deg[i] = _sc_embed_deg(cat_b[sl[i]], emb_all, dst_b[sl[i]], th)
    xcat[i] = x4_flat.reshape(th, NP, 64)
  for i in (0, 1):
    y1[i] = _mm1(xcat[i], num_p[sl[i]], deg[i], W1, th)
  for i in (0, 1):
    scat1[i] = _sc_scatter(y1[i], src_b[sl[i]], dst_b[sl[i]], zconv, 4, th)
    y2[i] = _post1_mm2(scat1[i], y1[i], deg[i], W2, b1.reshape(1, 128), th)
  for i in (0, 1):
    scat2[i] = _sc_scatter(y2[i], src_b[sl[i]], dst_b[sl[i]], zconv, 2, th)
  h = _post2_gru(scat2[0], scat2[1], y2[0], y2[1], deg[0], deg[1],
                 b2.reshape(1, 64), W_ih.T, W_hh.T,
                 b_ih.reshape(1, 192), b_hh.reshape(1, 192), th)
  return h[:N]


# speedup vs baseline: 1.0560x; 1.0029x over previous
import functools

import jax
import jax.numpy as jnp
from jax import lax
from jax.experimental import pallas as pl
from jax.experimental.pallas import tpu as pltpu
from jax.experimental.pallas import tpu_sc as plsc

T = 12
N = 50000
E = 800000
NP = 50176
RT = NP // 16
EP = 802816
EC = EP // 128
CAT_C = 4 * NP // 128
BN = 256
NB = NP // BN

_mesh = plsc.VectorSubcoreMesh(core_axis_name="c", subcore_axis_name="s")
_sc_params = pltpu.CompilerParams(use_tc_tiling_on_sc=False,
                                  needs_layout_passes=False)


def _sc_embed_deg(cat_b, emb_all, dst_b, th):
  DG = 7

  @functools.partial(
      pl.kernel,
      out_type=[
          jax.ShapeDtypeStruct((NP * 4 * th, 16), jnp.float32),
          jax.ShapeDtypeStruct((th, 32, NP), jnp.float32),
      ],
      mesh=_mesh,
      scratch_types=[
          pltpu.VMEM((2, 7, 128), jnp.int32),
          pltpu.VMEM((2, 7, 128, 16), jnp.float32),
          pltpu.VMEM((DG, 128), jnp.int32),
          pltpu.VMEM((NP,), jnp.float32),
          [pltpu.SemaphoreType.DMA] * 2,
          [pltpu.SemaphoreType.DMA] * 2,
      ],
      compiler_params=_sc_params,
  )
  def k(cat_hbm, emb_hbm, dst_hbm, x4_out, deg_out,
        eidx_v, erows_v, didx_v, dloc_v, semg, semw):
    c = lax.axis_index("c")
    s = lax.axis_index("s")
    w = c * 16 + s
    ones16 = jnp.full((16,), 1.0, jnp.float32)
    zeros16 = jnp.zeros((16,), jnp.float32)
    emb_chunks = CAT_C // 32
    deg_chunks = EC // 32
    EG = 7

    def e_load_fire(t, g, b):
      base = w * emb_chunks + g * EG
      pltpu.sync_copy(cat_hbm.at[t, pl.ds(base, EG)], eidx_v.at[b])
      for j in range(EG):
        pltpu.async_copy(emb_hbm.at[eidx_v.at[b, j]], erows_v.at[b, j],
                         semg[b])

    def e_gather_to_wb(t, g, b):
      base = w * emb_chunks + g * EG
      for j in range(EG):
        pltpu.make_async_copy(emb_hbm.at[eidx_v.at[b, j]], erows_v.at[b, j],
                              semg[b]).wait()
      for j in range(EG):
        pltpu.async_copy(erows_v.at[b, j],
                         x4_out.at[pl.ds((t * CAT_C + base + j) * 128, 128)],
                         semw[b])

    def e_drain_wb(t, g, b):
      base = w * emb_chunks + g * EG
      for j in range(EG):
        pltpu.make_async_copy(erows_v.at[b, j],
                              x4_out.at[pl.ds((t * CAT_C + base + j) * 128,
                                              128)],
                              semw[b]).wait()

    for t in range(th):
      for b in range(2):
        e_load_fire(t, b, b)

      @pl.loop(0, emb_chunks // EG, step=2, unroll=False)
      def _(i):
        for b in range(2):
          @pl.when(i + b < emb_chunks // EG)
          def _():
            e_gather_to_wb(t, i + b, b)

        for b in range(2):
          @pl.when(i + b + 2 < emb_chunks // EG)
          def _():
            e_drain_wb(t, i + b, b)
            e_load_fire(t, i + b + 2, b)

      for b in range(2):
        e_drain_wb(t, 0, b)

      @pl.loop(0, NP // 16)
      def _(r):
        dloc_v[pl.ds(r * 16, 16)] = zeros16

      @pl.loop(0, deg_chunks // DG)
      def _(g):
        ch = w * deg_chunks + g * DG
        pltpu.sync_copy(dst_hbm.at[t, pl.ds(ch, DG)], didx_v)
        for jj in range(DG):
          for kk in range(8):
            idx = didx_v[jj, pl.ds(kk * 16, 16)]
            plsc.addupdate_scatter(dloc_v, [idx], ones16)

      pltpu.sync_copy(dloc_v, deg_out.at[t, w])

  return k(cat_b, emb_all, dst_b)


def _sc_scatter(y_b, src_b, dst_b, zconv, cb_total, th):
  passes = cb_total // 2
  conv_chunks = EC // 16
  G = 2
  NBUF = 3
  GROUPS = conv_chunks // G

  @functools.partial(
      pl.kernel,
      out_type=jax.ShapeDtypeStruct((th, cb_total, NP, 32), jnp.float32),
      mesh=_mesh,
      scratch_types=[
          pltpu.VMEM((NBUF, G, 128), jnp.int32),
          pltpu.VMEM((NBUF, G, 128), jnp.int32),
          pltpu.VMEM((NBUF, G, 128, 32), jnp.float32),
          pltpu.VMEM_SHARED((NP, 32), jnp.float32),
          [pltpu.SemaphoreType.DMA] * NBUF,
          [pltpu.SemaphoreType.DMA] * NBUF,
      ],
      compiler_params=_sc_params,
  )
  def k(y_hbm, src_hbm, dst_hbm, zc_hbm, scat_out, sidx_v, didx_v, rows_v,
        acc_sh, semg, sems):
    c = lax.axis_index("c")
    s = lax.axis_index("s")

    def load_and_fire(t, cb, g, b):
      base = s * conv_chunks + g * G
      pltpu.sync_copy(src_hbm.at[t, pl.ds(base, G)], sidx_v.at[b])
      pltpu.sync_copy(dst_hbm.at[t, pl.ds(base, G)], didx_v.at[b])
      for j in range(G):
        pltpu.async_copy(y_hbm.at[t, cb].at[sidx_v.at[b, j]],
                         rows_v.at[b, j], semg[b])

    def gather_to_scatter(t, cb, b):
      for j in range(G):
        pltpu.make_async_copy(y_hbm.at[t, cb].at[sidx_v.at[b, j]],
                              rows_v.at[b, j], semg[b]).wait()
      for j in range(G):
        pltpu.async_copy(rows_v.at[b, j], acc_sh.at[didx_v.at[b, j]],
                         sems[b], add=True)

    def drain_scatter(b):
      for j in range(G):
        pltpu.make_async_copy(rows_v.at[b, j], acc_sh.at[didx_v.at[b, j]],
                              sems[b]).wait()

    for t in range(th):
      for p in range(passes):
        cb = c * passes + p
        pltpu.sync_copy(zc_hbm, acc_sh.at[pl.ds(s * RT, RT)])
        plsc.subcore_barrier()

        for b in range(NBUF):
          load_and_fire(t, cb, b, b)

        @pl.loop(0, GROUPS, step=NBUF)
        def _(i):
          for b in range(NBUF):
            @pl.when(i + b < GROUPS)
            def _():
              gather_to_scatter(t, cb, b)

          for b in range(NBUF):
            @pl.when(i + b + NBUF < GROUPS)
            def _():
              drain_scatter(b)
              load_and_fire(t, cb, i + b + NBUF, b)

        for b in range(NBUF):
          drain_scatter(b)
        plsc.subcore_barrier()
        pltpu.sync_copy(acc_sh.at[pl.ds(s * RT, RT)],
                        scat_out.at[t, cb, pl.ds(s * RT, RT)])

  return k(y_b, src_b, dst_b, zconv)


def _mm1(xcat, num_p, deg2, W1, th):
  def body(xcat_ref, num_ref, deg_ref, w1_ref, y1_ref):
    deg = jnp.sum(deg_ref[0], axis=0) + 1.0
    dis = lax.rsqrt(deg)
    xw = jnp.dot(xcat_ref[0], w1_ref[0:64],
                 preferred_element_type=jnp.float32)
    xw += jnp.dot(num_ref[0], w1_ref[64:96],
                  preferred_element_type=jnp.float32)
    y = xw * dis[:, None]
    for cb in range(4):
      y1_ref[0, cb] = y[:, 32 * cb:32 * (cb + 1)]

  return pl.pallas_call(
      body,
      grid=(th, NB),
      in_specs=[
          pl.BlockSpec((1, BN, 64), lambda t, n: (t, n, 0)),
          pl.BlockSpec((1, BN, 32), lambda t, n: (t, n, 0)),
          pl.BlockSpec((1, 32, BN), lambda t, n: (t, 0, n)),
          pl.BlockSpec((96, 128), lambda t, n: (0, 0)),
      ],
      out_specs=pl.BlockSpec((1, 4, BN, 32), lambda t, n: (t, 0, n, 0)),
      out_shape=jax.ShapeDtypeStruct((th, 4, NP, 32), jnp.float32),
      compiler_params=pltpu.CompilerParams(
          dimension_semantics=("parallel", "parallel")),
  )(xcat, num_p, deg2, W1)


def _post1_mm2(scat1, y1, deg2, W2, b1, th):
  def body(scat_ref, y1_ref, deg_ref, w2_ref, b1_ref, y2_ref):
    deg = jnp.sum(deg_ref[0], axis=0) + 1.0
    dis = lax.rsqrt(deg)[:, None]
    h = jnp.concatenate(
        [scat_ref[0, i] + y1_ref[0, i] for i in range(4)], axis=1)
    h1 = jnp.maximum(h * dis + b1_ref[0], 0.0)
    y2 = jnp.dot(h1, w2_ref[...], preferred_element_type=jnp.float32) * dis
    for i in range(2):
      y2_ref[0, i] = y2[:, 32 * i:32 * (i + 1)]

  return pl.pallas_call(
      body,
      grid=(th, NB),
      in_specs=[
          pl.BlockSpec((1, 4, BN, 32), lambda t, n: (t, 0, n, 0)),
          pl.BlockSpec((1, 4, BN, 32), lambda t, n: (t, 0, n, 0)),
          pl.BlockSpec((1, 32, BN), lambda t, n: (t, 0, n)),
          pl.BlockSpec((128, 64), lambda t, n: (0, 0)),
          pl.BlockSpec((1, 128), lambda t, n: (0, 0)),
      ],
      out_specs=pl.BlockSpec((1, 2, BN, 32), lambda t, n: (t, 0, n, 0)),
      out_shape=jax.ShapeDtypeStruct((th, 2, NP, 32), jnp.float32),
      compiler_params=pltpu.CompilerParams(
          dimension_semantics=("parallel", "parallel")),
  )(scat1, y1, deg2, W2, b1)


def _post2_gru(scat2a, scat2b, y2a, y2b, deg2a, deg2b, b2, W_ihT, W_hhT,
               b_ih, b_hh, th):
  def body(scat_a, scat_b, y2_a, y2_b, deg_a, deg_b, b2_ref, wih_ref,
           whh_ref, bih_ref, bhh_ref, h_ref):
    h = jnp.zeros((BN, 64), jnp.float32)
    for t in range(2 * th):
      scat_ref, y2_ref, deg_ref = (
          (scat_a, y2_a, deg_a) if t < th else (scat_b, y2_b, deg_b))
      tt = t if t < th else t - th
      deg = jnp.sum(deg_ref[tt], axis=0) + 1.0
      dis = lax.rsqrt(deg)[:, None]
      x = jnp.concatenate(
          [scat_ref[tt, i] + y2_ref[tt, i] for i in range(2)], axis=1)
      x = x * dis + b2_ref[0]
      gi = jnp.dot(x, wih_ref[...],
                   preferred_element_type=jnp.float32) + bih_ref[0]
      gh = jnp.dot(h, whh_ref[...],
                   preferred_element_type=jnp.float32) + bhh_ref[0]
      r = jax.nn.sigmoid(gi[:, 0:64] + gh[:, 0:64])
      z = jax.nn.sigmoid(gi[:, 64:128] + gh[:, 64:128])
      n_ = jnp.tanh(gi[:, 128:192] + r * gh[:, 128:192])
      h = (1.0 - z) * n_ + z * h
    h_ref[...] = h

  half = [
      pl.BlockSpec((th, 2, BN, 32), lambda n: (0, 0, n, 0)),
      pl.BlockSpec((th, 2, BN, 32), lambda n: (0, 0, n, 0)),
  ]
  return pl.pallas_call(
      body,
      grid=(NB,),
      in_specs=half + half[:1] * 0 + [
          pl.BlockSpec((th, 2, BN, 32), lambda n: (0, 0, n, 0)),
          pl.BlockSpec((th, 2, BN, 32), lambda n: (0, 0, n, 0)),
          pl.BlockSpec((th, 32, BN), lambda n: (0, 0, n)),
          pl.BlockSpec((th, 32, BN), lambda n: (0, 0, n)),
          pl.BlockSpec((1, 64), lambda n: (0, 0)),
          pl.BlockSpec((64, 192), lambda n: (0, 0)),
          pl.BlockSpec((64, 192), lambda n: (0, 0)),
          pl.BlockSpec((1, 192), lambda n: (0, 0)),
          pl.BlockSpec((1, 192), lambda n: (0, 0)),
      ],
      out_specs=pl.BlockSpec((BN, 64), lambda n: (n, 0)),
      out_shape=jax.ShapeDtypeStruct((NP, 64), jnp.float32),
      compiler_params=pltpu.CompilerParams(
          dimension_semantics=("parallel",)),
  )(scat2a, scat2b, y2a, y2b, deg2a, deg2b, b2, W_ihT, W_hhT, b_ih, b_hh)


def kernel(cat_x, num_x, edges, emb0, emb1, emb2, emb3, W1, b1, W2, b2,
           W_ih, W_hh, b_ih, b_hh):
  f32 = jnp.float32
  i32 = jnp.int32

  emb_all = jnp.concatenate([emb0, emb1, emb2, emb3], axis=0)
  offs = jnp.array([0, 1000, 2000, 3000], i32)
  cat_o = cat_x + offs[None, None, :]
  cat_o = jnp.pad(cat_o, ((0, 0), (0, NP - N), (0, 0)))
  cat_b = cat_o.reshape(T, CAT_C, 128)

  pad_src = jnp.broadcast_to(
      (jnp.arange(EP - E) % 128).astype(i32), (T, EP - E))
  pad_dst = jnp.broadcast_to(
      (N + jnp.arange(EP - E) % (NP - N)).astype(i32), (T, EP - E))
  src_b = jnp.concatenate([edges[:, 0], pad_src], axis=1).reshape(T, EC, 128)
  dst_b = jnp.concatenate([edges[:, 1], pad_dst], axis=1).reshape(T, EC, 128)

  num_p = jnp.pad(num_x, ((0, 0), (0, NP - N), (0, 0)))
  zconv = jnp.zeros((RT, 32), f32)

  th = T // 2
  xcat, y1, scat1, y2, scat2, deg = {}, {}, {}, {}, {}, {}
  sl = {0: slice(0, th), 1: slice(th, T)}
  for i in (0, 1):
    x4_flat, deg[i] = _sc_embed_deg(cat_b[sl[i]], emb_all, dst_b[sl[i]], th)
    xcat[i] = x4_flat.reshape(th, NP, 64)
  for i in (0, 1):
    y1[i] = _mm1(xcat[i], num_p[sl[i]], deg[i], W1, th)
  for i in (0, 1):
    scat1[i] = _sc_scatter(y1[i], src_b[sl[i]], dst_b[sl[i]], zconv, 4, th)
    y2[i] = _post1_mm2(scat1[i], y1[i], deg[i], W2, b1.reshape(1, 128), th)
  for i in (0, 1):
    scat2[i] = _sc_scatter(y2[i], src_b[sl[i]], dst_b[sl[i]], zconv, 2, th)
  h = _post2_gru(scat2[0], scat2[1], y2[0], y2[1], deg[0], deg[1],
                 b2.reshape(1, 64), W_ih.T, W_hh.T,
                 b_ih.reshape(1, 192), b_hh.reshape(1, 192), th)
  return h[:N]
